# baseline (speedup 1.0000x reference)
import functools

import jax
import jax.numpy as jnp
from jax import lax
from jax.experimental import pallas as pl
from jax.experimental.pallas import tpu as pltpu
from jax.experimental.pallas import tpu_sc as plsc

B = 16384
NUM_NUM = 13
NUM_CAT = 26
CARD = 100000
STATE = 32
NF = NUM_NUM + NUM_CAT

NC = 2
NS = 16
L = 16
NW = NC * NS
BPW = B // NW
R = 32
NCHUNK = BPW // R

mesh = plsc.VectorSubcoreMesh(core_axis_name="c", subcore_axis_name="s")


@functools.partial(
    pl.kernel,
    out_type=jax.ShapeDtypeStruct((B * NF, STATE), jnp.float32),
    mesh=mesh,
    compiler_params=pltpu.CompilerParams(use_tc_tiling_on_sc=False),
    scratch_types=[
        pltpu.VMEM((NUM_CAT * R,), jnp.int32),
        pltpu.VMEM((L * R,), jnp.float32),
        pltpu.VMEM((NUM_NUM * STATE,), jnp.float32),
        pltpu.VMEM((NUM_NUM * STATE,), jnp.float32),
        pltpu.VMEM((NUM_CAT * R, STATE), jnp.float32),
        pltpu.VMEM((NF * R, STATE), jnp.float32),
        pltpu.SemaphoreType.DMA,
    ],
)
def _sc_embed(xnum_hbm, xcat_hbm, w_hbm, b_hbm, tab_hbm, out_hbm,
              idxc, xnum_v, wv, bv, cat_v, out_v, sem):
    wid = lax.axis_index("s") * NC + lax.axis_index("c")
    base = wid * BPW

    pltpu.sync_copy(w_hbm, wv)
    pltpu.sync_copy(b_hbm, bv)

    def chunk_body(k, carry):
        r0 = base + k * R

        pltpu.sync_copy(xcat_hbm.at[pl.ds(NUM_CAT * r0, NUM_CAT * R)], idxc)
        pltpu.sync_copy(xnum_hbm.at[pl.ds(L * r0, L * R)], xnum_v)

        for q in range(NUM_CAT * R // L):
            offs = ((q * L + lax.iota(jnp.int32, L)) % NUM_CAT) * CARD
            idxc[pl.ds(q * L, L)] = idxc[pl.ds(q * L, L)] + offs

        gather = pltpu.async_copy(tab_hbm.at[idxc], cat_v, sem)

        def num_body(r, c):
            xrow = xnum_v[pl.ds(L * r, L)]
            for i in range(NUM_NUM):
                xb = lax.gather(
                    xrow, jnp.full((L, 1), i, jnp.int32),
                    lax.GatherDimensionNumbers(
                        offset_dims=(), collapsed_slice_dims=(0,),
                        start_index_map=(0,)),
                    slice_sizes=(1,),
                    mode=lax.GatherScatterMode.PROMISE_IN_BOUNDS)
                for h in range(STATE // L):
                    w16 = wv[pl.ds(i * STATE + h * L, L)]
                    b16 = bv[pl.ds(i * STATE + h * L, L)]
                    out_v[NF * r + i, pl.ds(h * L, L)] = xb * w16 + b16
            return c
        lax.fori_loop(0, R, num_body, 0)

        gather.wait()

        def asm_body(r, c):
            for t in range(NUM_CAT):
                for h in range(STATE // L):
                    out_v[NF * r + NUM_NUM + t, pl.ds(h * L, L)] = (
                        cat_v[NUM_CAT * r + t, pl.ds(h * L, L)])
            return c
        lax.fori_loop(0, R, asm_body, 0)

        pltpu.sync_copy(out_v, out_hbm.at[pl.ds(NF * r0, NF * R)])
        return carry

    lax.fori_loop(0, NCHUNK, chunk_body, 0)


def kernel(x_numeric, x_categorical, W_num, b_num, tables):
    xnum_flat = jnp.pad(x_numeric, ((0, 0), (0, L - NUM_NUM))).reshape(-1)
    xcat_flat = x_categorical.astype(jnp.int32).reshape(-1)
    w_flat = W_num.reshape(-1)
    b_flat = b_num.reshape(-1)
    tab_flat = tables.reshape(-1, STATE)
    out = _sc_embed(xnum_flat, xcat_flat, w_flat, b_flat, tab_flat)
    return out.reshape(B, NF * STATE)


# speedup vs baseline: 3.0716x; 3.0716x over previous
import functools

import jax
import jax.numpy as jnp
from jax import lax
from jax.experimental import pallas as pl
from jax.experimental.pallas import tpu as pltpu
from jax.experimental.pallas import tpu_sc as plsc

B = 16384
NUM_NUM = 13
NUM_CAT = 26
CARD = 100000
STATE = 32
NF = NUM_NUM + NUM_CAT

NC = 2
NS = 16
L = 16
NW = NC * NS
BH = B // 2

mesh = plsc.VectorSubcoreMesh(core_axis_name="c", subcore_axis_name="s")


@functools.partial(
    pl.kernel,
    out_type=jax.ShapeDtypeStruct((NF * STATE, B), jnp.float32),
    mesh=mesh,
    compiler_params=pltpu.CompilerParams(use_tc_tiling_on_sc=True,
                                         needs_layout_passes=False),
    scratch_types=[
        pltpu.VMEM((1, CARD), jnp.float32),
        pltpu.VMEM((1, BH), jnp.int32),
        pltpu.VMEM((1, BH), jnp.float32),
        pltpu.VMEM((1, L), jnp.float32),
        pltpu.VMEM((1, L), jnp.float32),
        pltpu.SemaphoreType.DMA,
    ],
)
def _sc_embed(tab_hbm, xcat_hbm, xnum_hbm, wt_hbm, bt_hbm, out_hbm,
              row_v, idx_v, out_v, w_v, b_v, sem):
    w = lax.axis_index("s") * NC + lax.axis_index("c")

    pltpu.sync_copy(wt_hbm.at[pl.ds(w, 1), :], w_v)
    pltpu.sync_copy(bt_hbm.at[pl.ds(w, 1), :], b_v)

    for k in range(NUM_NUM):
        lane = jnp.full((L, 1), k, jnp.int32)
        dn = lax.GatherDimensionNumbers(
            offset_dims=(), collapsed_slice_dims=(0,), start_index_map=(0,))
        wk = lax.gather(w_v[0, :], lane, dn, slice_sizes=(1,),
                        mode=lax.GatherScatterMode.PROMISE_IN_BOUNDS)
        bk = lax.gather(b_v[0, :], lane, dn, slice_sizes=(1,),
                        mode=lax.GatherScatterMode.PROMISE_IN_BOUNDS)
        for hh in range(B // BH):
            pltpu.sync_copy(xnum_hbm.at[pl.ds(k, 1), pl.ds(hh * BH, BH)], out_v)

            def num_body(q, c, wk=wk, bk=bk):
                out_v[0, pl.ds(q * L, L)] = out_v[0, pl.ds(q * L, L)] * wk + bk
                return c
            lax.fori_loop(0, BH // L, num_body, 0)
            pltpu.sync_copy(out_v, out_hbm.at[pl.ds(STATE * k + w, 1),
                                              pl.ds(hh * BH, BH)])

    def cat_body(i, c):
        pltpu.sync_copy(tab_hbm.at[i, pl.ds(w, 1), :], row_v)
        for hh in range(B // BH):
            pltpu.sync_copy(xcat_hbm.at[pl.ds(i, 1), pl.ds(hh * BH, BH)], idx_v)

            def gat_body(q, c2):
                idx16 = idx_v[0, pl.ds(q * L, L)]
                out_v[0, pl.ds(q * L, L)] = plsc.load_gather(
                    row_v, [jnp.zeros((L,), jnp.int32), idx16])
                return c2
            lax.fori_loop(0, BH // L, gat_body, 0)
            pltpu.sync_copy(out_v,
                            out_hbm.at[pl.ds(STATE * (NUM_NUM + i) + w, 1),
                                       pl.ds(hh * BH, BH)])
        return c
    lax.fori_loop(0, NUM_CAT, cat_body, 0)


def kernel(x_numeric, x_categorical, W_num, b_num, tables):
    tab_t = tables.transpose(0, 2, 1)
    xcat_t = x_categorical.astype(jnp.int32).T
    xnum_t = x_numeric.T
    w_t = jnp.pad(W_num.T, ((0, 0), (0, L - NUM_NUM)))
    b_t = jnp.pad(b_num.T, ((0, 0), (0, L - NUM_NUM)))
    out_t = _sc_embed(tab_t, xcat_t, xnum_t, w_t, b_t)
    return out_t.T


# speedup vs baseline: 4.8852x; 1.5904x over previous
import functools

import jax
import jax.numpy as jnp
from jax import lax
from jax.experimental import pallas as pl
from jax.experimental.pallas import tpu as pltpu
from jax.experimental.pallas import tpu_sc as plsc

B = 16384
NUM_NUM = 13
NUM_CAT = 26
CARD = 100000
STATE = 32
NF = NUM_NUM + NUM_CAT

NC = 2
NS = 16
L = 16
NW = NC * NS
BH = B // 4
NCHUNK = B // BH

mesh = plsc.VectorSubcoreMesh(core_axis_name="c", subcore_axis_name="s")


@functools.partial(
    pl.kernel,
    out_type=jax.ShapeDtypeStruct((NF * STATE, B), jnp.float32),
    mesh=mesh,
    compiler_params=pltpu.CompilerParams(use_tc_tiling_on_sc=True,
                                         needs_layout_passes=False),
    scratch_types=[
        pltpu.VMEM((1, CARD), jnp.float32),
        pltpu.VMEM((2, BH), jnp.int32),
        pltpu.VMEM((2, BH), jnp.float32),
        pltpu.VMEM((1, L), jnp.float32),
        pltpu.VMEM((1, L), jnp.float32),
        pltpu.SemaphoreType.DMA,
        pltpu.SemaphoreType.DMA,
        pltpu.SemaphoreType.DMA,
        pltpu.SemaphoreType.DMA,
    ],
)
def _sc_embed(tab_hbm, xcat_hbm, xnum_hbm, wt_hbm, bt_hbm, out_hbm,
              row_v, idx_b, out_b, w_v, b_v, sem_i0, sem_i1, sem_o0, sem_o1):
    sem_i = (sem_i0, sem_i1)
    sem_o = (sem_o0, sem_o1)
    w = lax.axis_index("s") * NC + lax.axis_index("c")

    pltpu.sync_copy(wt_hbm.at[pl.ds(w, 1), :], w_v)
    pltpu.sync_copy(bt_hbm.at[pl.ds(w, 1), :], b_v)

    for k in range(NUM_NUM):
        lane = jnp.full((L, 1), k, jnp.int32)
        dn = lax.GatherDimensionNumbers(
            offset_dims=(), collapsed_slice_dims=(0,), start_index_map=(0,))
        wk = lax.gather(w_v[0, :], lane, dn, slice_sizes=(1,),
                        mode=lax.GatherScatterMode.PROMISE_IN_BOUNDS)
        bk = lax.gather(b_v[0, :], lane, dn, slice_sizes=(1,),
                        mode=lax.GatherScatterMode.PROMISE_IN_BOUNDS)
        for hh in range(NCHUNK):
            hb = hh % 2
            pltpu.sync_copy(xnum_hbm.at[pl.ds(k, 1), pl.ds(hh * BH, BH)],
                            out_b.at[pl.ds(hb, 1), :])

            @plsc.parallel_loop(0, BH, step=L, unroll=8)
            def num_body(p, wk=wk, bk=bk, hb=hb):
                out_b[hb, pl.ds(p, L)] = out_b[hb, pl.ds(p, L)] * wk + bk
            pltpu.sync_copy(out_b.at[pl.ds(hb, 1), :],
                            out_hbm.at[pl.ds(STATE * k + w, 1),
                                       pl.ds(hh * BH, BH)])

    def fire_idx(i, hh, hb):
        pltpu.async_copy(xcat_hbm.at[pl.ds(i, 1), pl.ds(hh * BH, BH)],
                         idx_b.at[pl.ds(hb, 1), :], sem_i[hb])

    def drain_idx(hb):
        pltpu.make_async_copy(xcat_hbm.at[pl.ds(0, 1), pl.ds(0, BH)],
                              idx_b.at[pl.ds(hb, 1), :], sem_i[hb]).wait()

    def drain_out(hb):
        pltpu.make_async_copy(xcat_hbm.at[pl.ds(0, 1), pl.ds(0, BH)],
                              out_b.at[pl.ds(hb, 1), :], sem_o[hb]).wait()

    fire_idx(0, 0, 0)

    def cat_body(i, c):
        pltpu.sync_copy(tab_hbm.at[i, pl.ds(w, 1), :], row_v)
        for hh in range(NCHUNK):
            hb = hh % 2
            drain_idx(hb)
            if hh + 1 < NCHUNK:
                fire_idx(i, hh + 1, (hb + 1) % 2)
            else:
                @pl.when(i + 1 < NUM_CAT)
                def _():
                    fire_idx(i + 1, 0, (hb + 1) % 2)
            if hh >= 2:
                drain_out(hb)
            else:
                @pl.when(i > 0)
                def _(hb=hb):
                    drain_out(hb)

            @plsc.parallel_loop(0, BH, step=L, unroll=8)
            def gat_body(p, hb=hb):
                idx16 = idx_b[hb, pl.ds(p, L)]
                out_b[hb, pl.ds(p, L)] = plsc.load_gather(
                    row_v, [jnp.zeros((L,), jnp.int32), idx16])
            pltpu.async_copy(
                out_b.at[pl.ds(hb, 1), :],
                out_hbm.at[pl.ds(STATE * (NUM_NUM + i) + w, 1),
                           pl.ds(hh * BH, BH)], sem_o[hb])
        return c
    lax.fori_loop(0, NUM_CAT, cat_body, 0)
    drain_out(0)
    drain_out(1)


def kernel(x_numeric, x_categorical, W_num, b_num, tables):
    tab_t = tables.transpose(0, 2, 1)
    xcat_t = x_categorical.astype(jnp.int32).T
    xnum_t = x_numeric.T
    w_t = jnp.pad(W_num.T, ((0, 0), (0, L - NUM_NUM)))
    b_t = jnp.pad(b_num.T, ((0, 0), (0, L - NUM_NUM)))
    out_t = _sc_embed(tab_t, xcat_t, xnum_t, w_t, b_t)
    return out_t.T


# speedup vs baseline: 4.9031x; 1.0037x over previous
import functools

import jax
import jax.numpy as jnp
from jax import lax
from jax.experimental import pallas as pl
from jax.experimental.pallas import tpu as pltpu
from jax.experimental.pallas import tpu_sc as plsc

B = 16384
NUM_NUM = 13
NUM_CAT = 26
CARD = 100000
STATE = 32
NF = NUM_NUM + NUM_CAT

NC = 2
NS = 16
L = 16
NW = NC * NS
BH = B // 2

mesh = plsc.VectorSubcoreMesh(core_axis_name="c", subcore_axis_name="s")


@functools.partial(
    pl.kernel,
    out_type=jax.ShapeDtypeStruct((NF * STATE, B), jnp.float32),
    mesh=mesh,
    compiler_params=pltpu.CompilerParams(use_tc_tiling_on_sc=True,
                                         needs_layout_passes=False),
    scratch_types=[
        pltpu.VMEM((1, CARD), jnp.float32),
        pltpu.VMEM((1, BH), jnp.int32),
        pltpu.VMEM((1, BH), jnp.float32),
        pltpu.VMEM((1, L), jnp.float32),
        pltpu.VMEM((1, L), jnp.float32),
        pltpu.SemaphoreType.DMA,
    ],
)
def _sc_embed(tab_hbm, xcat_hbm, xnum_hbm, wt_hbm, bt_hbm, out_hbm,
              row_v, idx_v, out_v, w_v, b_v, sem):
    w = lax.axis_index("s") * NC + lax.axis_index("c")

    pltpu.sync_copy(wt_hbm.at[pl.ds(w, 1), :], w_v)
    pltpu.sync_copy(bt_hbm.at[pl.ds(w, 1), :], b_v)

    for k in range(NUM_NUM):
        lane = jnp.full((L, 1), k, jnp.int32)
        dn = lax.GatherDimensionNumbers(
            offset_dims=(), collapsed_slice_dims=(0,), start_index_map=(0,))
        wk = lax.gather(w_v[0, :], lane, dn, slice_sizes=(1,),
                        mode=lax.GatherScatterMode.PROMISE_IN_BOUNDS)
        bk = lax.gather(b_v[0, :], lane, dn, slice_sizes=(1,),
                        mode=lax.GatherScatterMode.PROMISE_IN_BOUNDS)
        for hh in range(B // BH):
            pltpu.sync_copy(xnum_hbm.at[pl.ds(k, 1), pl.ds(hh * BH, BH)], out_v)

            @plsc.parallel_loop(0, BH, step=L, unroll=8)
            def num_body(p, wk=wk, bk=bk):
                out_v[0, pl.ds(p, L)] = out_v[0, pl.ds(p, L)] * wk + bk
            pltpu.sync_copy(out_v, out_hbm.at[pl.ds(STATE * k + w, 1),
                                              pl.ds(hh * BH, BH)])

    def cat_body(i, c):
        pltpu.sync_copy(tab_hbm.at[i, pl.ds(w, 1), :], row_v)
        for hh in range(B // BH):
            pltpu.sync_copy(xcat_hbm.at[pl.ds(i, 1), pl.ds(hh * BH, BH)], idx_v)

            @plsc.parallel_loop(0, BH, step=L, unroll=8)
            def gat_body(p):
                idx16 = idx_v[0, pl.ds(p, L)]
                out_v[0, pl.ds(p, L)] = plsc.load_gather(
                    row_v, [jnp.zeros((L,), jnp.int32), idx16])
            pltpu.sync_copy(out_v,
                            out_hbm.at[pl.ds(STATE * (NUM_NUM + i) + w, 1),
                                       pl.ds(hh * BH, BH)])
        return c
    lax.fori_loop(0, NUM_CAT, cat_body, 0)


def kernel(x_numeric, x_categorical, W_num, b_num, tables):
    tab_t = tables.transpose(0, 2, 1)
    xcat_t = x_categorical.astype(jnp.int32).T
    xnum_t = x_numeric.T
    w_t = jnp.pad(W_num.T, ((0, 0), (0, L - NUM_NUM)))
    b_t = jnp.pad(b_num.T, ((0, 0), (0, L - NUM_NUM)))
    out_t = _sc_embed(tab_t, xcat_t, xnum_t, w_t, b_t)
    return out_t.T
